# baseline (speedup 1.0000x reference)
import functools

import jax
import jax.numpy as jnp
from jax import lax
from jax.experimental import pallas as pl
from jax.experimental.pallas import tpu as pltpu
from jax.experimental.pallas import tpu_sc as plsc

K = 20
EPS = 1e-5
NEG = -3e38


def _topk_body(xa_ref, xr_ref, out_ref):
    xa = xa_ref[0]
    xr = xr_ref[0]
    n = xa.shape[0]
    rb = xr.shape[0]
    inner = jax.lax.dot_general(
        xr.astype(jnp.bfloat16), xa.astype(jnp.bfloat16),
        (((1,), (1,)), ((), ())), preferred_element_type=jnp.float32)
    xxa = jnp.sum(xa * xa, axis=1)
    xxr = jnp.sum(xr * xr, axis=1)
    nd = (-xxr[:, None] - (-2.0 * inner)) - xxa[None, :]
    iota = jax.lax.broadcasted_iota(jnp.int32, (rb, n), 1)
    vals = nd
    cols = []
    for _ in range(K):
        m = jnp.max(vals, axis=1, keepdims=True)
        cand = jnp.where(vals == m, iota, n)
        fi = jnp.min(cand, axis=1, keepdims=True)
        cols.append(fi)
        vals = jnp.where(iota == fi, NEG, vals)
    out_ref[0] = jnp.concatenate(cols, axis=1)


def _knn_pallas(xn):
    B, N, C = xn.shape
    RB = 1024
    return pl.pallas_call(
        _topk_body,
        grid=(B, N // RB),
        in_specs=[
            pl.BlockSpec((1, N, C), lambda b, r: (b, 0, 0)),
            pl.BlockSpec((1, RB, C), lambda b, r: (b, r, 0)),
        ],
        out_specs=pl.BlockSpec((1, RB, K), lambda b, r: (b, r, 0)),
        out_shape=jax.ShapeDtypeStruct((B, N, K), jnp.int32),
    )(xn, xn)


def _sc_gather(x2d, gidx, ch):
    R, Cp = x2d.shape
    M = gidx.shape[0]
    info = plsc.get_sparse_core_info()
    nw = info.num_cores * info.num_subcores
    per_w = M // nw
    n_ch = per_w // ch
    mesh = plsc.VectorSubcoreMesh(core_axis_name="c", subcore_axis_name="s")

    @functools.partial(
        pl.kernel, mesh=mesh,
        out_type=jax.ShapeDtypeStruct((M, Cp), jnp.float32),
        scratch_types=[
            pltpu.VMEM((ch,), jnp.int32),
            pltpu.VMEM((ch,), jnp.int32),
            pltpu.VMEM((ch, Cp), jnp.float32),
            pltpu.VMEM((ch, Cp), jnp.float32),
            pltpu.SemaphoreType.DMA,
            pltpu.SemaphoreType.DMA,
            pltpu.SemaphoreType.DMA,
            pltpu.SemaphoreType.DMA,
        ],
    )
    def k(x_hbm, idx_hbm, out_hbm, i0, i1, r0, r1, sg0, sg1, sw0, sw1):
        wid = lax.axis_index("s") * info.num_cores + lax.axis_index("c")
        base = wid * per_w

        def body(t):
            off0 = base + (2 * t) * ch
            off1 = base + (2 * t + 1) * ch
            pltpu.sync_copy(idx_hbm.at[pl.ds(off0, ch)], i0)
            pltpu.sync_copy(idx_hbm.at[pl.ds(off1, ch)], i1)
            g0 = pltpu.async_copy(x_hbm.at[i0], r0, sg0)
            g1 = pltpu.async_copy(x_hbm.at[i1], r1, sg1)
            g0.wait()
            w0 = pltpu.async_copy(r0, out_hbm.at[pl.ds(off0, ch)], sw0)
            g1.wait()
            w1 = pltpu.async_copy(r1, out_hbm.at[pl.ds(off1, ch)], sw1)
            w0.wait()
            w1.wait()

        pl.loop(0, n_ch // 2)(body)

    return k(x2d, gidx)


def _edge_body(G_ref, x_ref, W_ref, M_ref, st_ref, s_acc, ss_acc):
    i = pl.program_id(0)
    j = pl.program_id(1)
    nb = pl.num_programs(1)

    @pl.when((i == 0) & (j == 0))
    def _init():
        s_acc[...] = jnp.zeros_like(s_acc)
        ss_acc[...] = jnp.zeros_like(ss_acc)

    x = x_ref[0]
    c = x.shape[1]
    Wb = W_ref[...].astype(jnp.bfloat16)
    mx = None
    s = None
    ss = None
    for kk in range(K):
        Gk = G_ref[0, :, kk, :c]
        f = jnp.concatenate([Gk - x, x], axis=1)
        h = jax.lax.dot_general(
            f.astype(jnp.bfloat16), Wb,
            (((1,), (1,)), ((), ())), preferred_element_type=jnp.float32)
        mx = h if mx is None else jnp.maximum(mx, h)
        hs = jnp.sum(h, axis=0, keepdims=True)
        hss = jnp.sum(h * h, axis=0, keepdims=True)
        s = hs if s is None else s + hs
        ss = hss if ss is None else ss + hss
    M_ref[0] = mx
    s_acc[...] += s
    ss_acc[...] += ss

    @pl.when((i == pl.num_programs(0) - 1) & (j == nb - 1))
    def _fin():
        st_ref[0:1] = s_acc[...]
        st_ref[1:2] = ss_acc[...]


def _edge_conv(G, xn, W):
    B, N, C = xn.shape
    Cp = G.shape[-1]
    O = W.shape[0]
    NB = 256
    return pl.pallas_call(
        _edge_body,
        grid=(B, N // NB),
        in_specs=[
            pl.BlockSpec((1, NB, K, Cp), lambda b, j: (b, j, 0, 0)),
            pl.BlockSpec((1, NB, C), lambda b, j: (b, j, 0)),
            pl.BlockSpec((O, 2 * C), lambda b, j: (0, 0)),
        ],
        out_specs=[
            pl.BlockSpec((1, NB, O), lambda b, j: (b, j, 0)),
            pl.BlockSpec((2, O), lambda b, j: (0, 0)),
        ],
        out_shape=[
            jax.ShapeDtypeStruct((B, N, O), jnp.float32),
            jax.ShapeDtypeStruct((2, O), jnp.float32),
        ],
        scratch_shapes=[
            pltpu.VMEM((1, O), jnp.float32),
            pltpu.VMEM((1, O), jnp.float32),
        ],
    )(G, xn, W)


def _fin_body(M_ref, st_ref, g_ref, b_ref, cnt_ref, o_ref):
    cnt = cnt_ref[0]
    mean = st_ref[0:1] / cnt
    var = st_ref[1:2] / cnt - mean * mean
    rs = jax.lax.rsqrt(var + EPS)
    xh = (M_ref[0] - mean) * rs * g_ref[0:1] + b_ref[0:1]
    o_ref[0] = jnp.where(xh >= 0, xh, 0.2 * xh)


def _finalize(M, st, g, b):
    B, N, O = M.shape
    NB = 512
    cnt = jnp.full((1,), float(B * N * K), jnp.float32)
    return pl.pallas_call(
        _fin_body,
        grid=(B, N // NB),
        in_specs=[
            pl.BlockSpec((1, NB, O), lambda bb, j: (bb, j, 0)),
            pl.BlockSpec((2, O), lambda bb, j: (0, 0)),
            pl.BlockSpec((1, O), lambda bb, j: (0, 0)),
            pl.BlockSpec((1, O), lambda bb, j: (0, 0)),
            pl.BlockSpec(memory_space=pltpu.SMEM),
        ],
        out_specs=pl.BlockSpec((1, NB, O), lambda bb, j: (bb, j, 0)),
        out_shape=jax.ShapeDtypeStruct((B, N, O), jnp.float32),
    )(M, st, g.reshape(1, O), b.reshape(1, O), cnt)


def _l5_body(cat_ref, W_ref, mx_ref, st_ref, s_acc, ss_acc, mx_acc):
    i = pl.program_id(0)
    j = pl.program_id(1)
    nb = pl.num_programs(1)

    @pl.when((i == 0) & (j == 0))
    def _init():
        s_acc[...] = jnp.zeros_like(s_acc)
        ss_acc[...] = jnp.zeros_like(ss_acc)

    @pl.when(j == 0)
    def _initmx():
        mx_acc[...] = jnp.full_like(mx_acc, NEG)

    h = jax.lax.dot_general(
        cat_ref[0].astype(jnp.bfloat16), W_ref[...].astype(jnp.bfloat16),
        (((1,), (1,)), ((), ())), preferred_element_type=jnp.float32)
    s_acc[...] += jnp.sum(h, axis=0, keepdims=True)
    ss_acc[...] += jnp.sum(h * h, axis=0, keepdims=True)
    mx_acc[...] = jnp.maximum(mx_acc[...], jnp.max(h, axis=0, keepdims=True))

    @pl.when(j == nb - 1)
    def _finmx():
        mx_ref[pl.ds(i, 1)] = mx_acc[...]

    @pl.when((i == pl.num_programs(0) - 1) & (j == nb - 1))
    def _fin():
        st_ref[0:1] = s_acc[...]
        st_ref[1:2] = ss_acc[...]


def _layer5(cat, W5):
    B, N, C = cat.shape
    O = W5.shape[0]
    NB = 512
    return pl.pallas_call(
        _l5_body,
        grid=(B, N // NB),
        in_specs=[
            pl.BlockSpec((1, NB, C), lambda b, j: (b, j, 0)),
            pl.BlockSpec((O, C), lambda b, j: (0, 0)),
        ],
        out_specs=[
            pl.BlockSpec((B, O), lambda b, j: (0, 0)),
            pl.BlockSpec((2, O), lambda b, j: (0, 0)),
        ],
        out_shape=[
            jax.ShapeDtypeStruct((B, O), jnp.float32),
            jax.ShapeDtypeStruct((2, O), jnp.float32),
        ],
        scratch_shapes=[
            pltpu.VMEM((1, O), jnp.float32),
            pltpu.VMEM((1, O), jnp.float32),
            pltpu.VMEM((1, O), jnp.float32),
        ],
    )(cat, W5)


def _emb_body(m5_ref, st_ref, g_ref, b_ref, cnt_ref, We_ref, o_ref):
    cnt = cnt_ref[0]
    mean = st_ref[0:1] / cnt
    var = st_ref[1:2] / cnt - mean * mean
    rs = jax.lax.rsqrt(var + EPS)
    xh = (m5_ref[...] - mean) * rs * g_ref[0:1] + b_ref[0:1]
    xg = jnp.where(xh >= 0, xh, 0.2 * xh)
    o_ref[...] = jax.lax.dot_general(
        xg.astype(jnp.bfloat16), We_ref[...].astype(jnp.bfloat16),
        (((1,), (1,)), ((), ())), preferred_element_type=jnp.float32)


def _embed(m5, st, g, b, Wemb, n_total):
    B, C = m5.shape
    O = Wemb.shape[0]
    cnt = jnp.full((1,), float(n_total), jnp.float32)
    return pl.pallas_call(
        _emb_body,
        in_specs=[
            pl.BlockSpec((B, C), lambda: (0, 0)),
            pl.BlockSpec((2, C), lambda: (0, 0)),
            pl.BlockSpec((1, C), lambda: (0, 0)),
            pl.BlockSpec((1, C), lambda: (0, 0)),
            pl.BlockSpec(memory_space=pltpu.SMEM),
            pl.BlockSpec((O, C), lambda: (0, 0)),
        ],
        out_specs=pl.BlockSpec((B, O), lambda: (0, 0)),
        out_shape=jax.ShapeDtypeStruct((B, O), jnp.float32),
    )(m5, st, g.reshape(1, C), b.reshape(1, C), cnt, Wemb)


def _edge_layer(xn, W, g, b):
    B, N, C = xn.shape
    idx = _knn_pallas(xn)
    gidx = (idx + (jnp.arange(B) * N)[:, None, None]).reshape(B * N * K)
    Cp = 128
    if C == Cp:
        x2d = xn.reshape(B * N, C)
    else:
        x2d = jnp.pad(xn.reshape(B * N, C), ((0, 0), (0, Cp - C)))
    G = _sc_gather(x2d, gidx, 256)
    M, st = _edge_conv(G.reshape(B, N, K, Cp), xn, W)
    return _finalize(M, st, g, b)


def kernel(x, W1, W2, W3, W4, W5, Wemb, g1, b1, g2, b2, g3, b3, g4, b4, g5, b5):
    x1 = _edge_layer(x, W1, g1, b1)
    x2 = _edge_layer(x1, W2, g2, b2)
    x3 = _edge_layer(x2, W3, g3, b3)
    x4 = _edge_layer(x3, W4, g4, b4)
    cat = jnp.concatenate([x1, x2, x3, x4], axis=-1)
    m5, st5 = _layer5(cat, W5)
    return _embed(m5, st5, g5, b5, Wemb, cat.shape[0] * cat.shape[1])


# speedup vs baseline: 1.2103x; 1.2103x over previous
import functools

import jax
import jax.numpy as jnp
from jax import lax
from jax.experimental import pallas as pl
from jax.experimental.pallas import tpu as pltpu
from jax.experimental.pallas import tpu_sc as plsc

K = 20
EPS = 1e-5
NEG = -3e38


def _topk_body(xa_ref, xr_ref, out_ref):
    xa = xa_ref[0]
    xr = xr_ref[0]
    n = xa.shape[0]
    rb = xr.shape[0]
    inner = jax.lax.dot_general(
        xr.astype(jnp.bfloat16), xa.astype(jnp.bfloat16),
        (((1,), (1,)), ((), ())), preferred_element_type=jnp.float32)
    xxa = jnp.sum(xa * xa, axis=1)
    xxr = jnp.sum(xr * xr, axis=1)
    nd = (-xxr[:, None] - (-2.0 * inner)) - xxa[None, :]
    iota = jax.lax.broadcasted_iota(jnp.int32, (rb, n), 1)
    vals = nd
    cols = []
    for _ in range(K):
        m = jnp.max(vals, axis=1, keepdims=True)
        cand = jnp.where(vals == m, iota, n)
        fi = jnp.min(cand, axis=1, keepdims=True)
        cols.append(fi)
        vals = jnp.where(iota == fi, NEG, vals)
    out_ref[0] = jnp.concatenate(cols, axis=1)


def _knn_pallas(xn):
    B, N, C = xn.shape
    RB = 512
    return pl.pallas_call(
        _topk_body,
        grid=(B, N // RB),
        in_specs=[
            pl.BlockSpec((1, N, C), lambda b, r: (b, 0, 0)),
            pl.BlockSpec((1, RB, C), lambda b, r: (b, r, 0)),
        ],
        out_specs=pl.BlockSpec((1, RB, K), lambda b, r: (b, r, 0)),
        out_shape=jax.ShapeDtypeStruct((B, N, K), jnp.int32),
    )(xn, xn)


def _sc_gather(x2d, gidx, ch):
    R, Cp = x2d.shape
    M = gidx.shape[0]
    info = plsc.get_sparse_core_info()
    nw = info.num_cores * info.num_subcores
    per_w = M // nw
    n_ch = per_w // ch
    mesh = plsc.VectorSubcoreMesh(core_axis_name="c", subcore_axis_name="s")

    @functools.partial(
        pl.kernel, mesh=mesh,
        out_type=jax.ShapeDtypeStruct((M, Cp), jnp.float32),
        scratch_types=[
            pltpu.VMEM((ch,), jnp.int32),
            pltpu.VMEM((ch,), jnp.int32),
            pltpu.VMEM((ch, Cp), jnp.float32),
            pltpu.VMEM((ch, Cp), jnp.float32),
            pltpu.SemaphoreType.DMA,
            pltpu.SemaphoreType.DMA,
            pltpu.SemaphoreType.DMA,
            pltpu.SemaphoreType.DMA,
        ],
    )
    def k(x_hbm, idx_hbm, out_hbm, i0, i1, r0, r1, sg0, sg1, sw0, sw1):
        wid = lax.axis_index("s") * info.num_cores + lax.axis_index("c")
        base = wid * per_w

        def body(t):
            off0 = base + (2 * t) * ch
            off1 = base + (2 * t + 1) * ch
            pltpu.sync_copy(idx_hbm.at[pl.ds(off0, ch)], i0)
            pltpu.sync_copy(idx_hbm.at[pl.ds(off1, ch)], i1)
            g0 = pltpu.async_copy(x_hbm.at[i0], r0, sg0)
            g1 = pltpu.async_copy(x_hbm.at[i1], r1, sg1)
            g0.wait()
            w0 = pltpu.async_copy(r0, out_hbm.at[pl.ds(off0, ch)], sw0)
            g1.wait()
            w1 = pltpu.async_copy(r1, out_hbm.at[pl.ds(off1, ch)], sw1)
            w0.wait()
            w1.wait()

        pl.loop(0, n_ch // 2)(body)

    return k(x2d, gidx)


def _edge_body(G_ref, x_ref, W_ref, M_ref, st_ref, s_acc, ss_acc):
    i = pl.program_id(0)
    j = pl.program_id(1)
    nb = pl.num_programs(1)

    @pl.when((i == 0) & (j == 0))
    def _init():
        s_acc[...] = jnp.zeros_like(s_acc)
        ss_acc[...] = jnp.zeros_like(ss_acc)

    x = x_ref[0]
    c = x.shape[1]
    Wb = W_ref[...].astype(jnp.bfloat16)
    mx = None
    s = None
    ss = None
    for kk in range(K):
        Gk = G_ref[0, :, kk, :c]
        f = jnp.concatenate([Gk - x, x], axis=1)
        h = jax.lax.dot_general(
            f.astype(jnp.bfloat16), Wb,
            (((1,), (1,)), ((), ())), preferred_element_type=jnp.float32)
        mx = h if mx is None else jnp.maximum(mx, h)
        hs = jnp.sum(h, axis=0, keepdims=True)
        hss = jnp.sum(h * h, axis=0, keepdims=True)
        s = hs if s is None else s + hs
        ss = hss if ss is None else ss + hss
    M_ref[0] = mx
    s_acc[...] += s
    ss_acc[...] += ss

    @pl.when((i == pl.num_programs(0) - 1) & (j == nb - 1))
    def _fin():
        st_ref[0:1] = s_acc[...]
        st_ref[1:2] = ss_acc[...]


def _edge_conv(G, xn, W):
    B, N, C = xn.shape
    Cp = G.shape[-1]
    O = W.shape[0]
    NB = 256
    return pl.pallas_call(
        _edge_body,
        grid=(B, N // NB),
        in_specs=[
            pl.BlockSpec((1, NB, K, Cp), lambda b, j: (b, j, 0, 0)),
            pl.BlockSpec((1, NB, C), lambda b, j: (b, j, 0)),
            pl.BlockSpec((O, 2 * C), lambda b, j: (0, 0)),
        ],
        out_specs=[
            pl.BlockSpec((1, NB, O), lambda b, j: (b, j, 0)),
            pl.BlockSpec((2, O), lambda b, j: (0, 0)),
        ],
        out_shape=[
            jax.ShapeDtypeStruct((B, N, O), jnp.float32),
            jax.ShapeDtypeStruct((2, O), jnp.float32),
        ],
        scratch_shapes=[
            pltpu.VMEM((1, O), jnp.float32),
            pltpu.VMEM((1, O), jnp.float32),
        ],
    )(G, xn, W)


def _fin_body(M_ref, st_ref, g_ref, b_ref, cnt_ref, o_ref):
    cnt = cnt_ref[0]
    mean = st_ref[0:1] / cnt
    var = st_ref[1:2] / cnt - mean * mean
    rs = jax.lax.rsqrt(var + EPS)
    xh = (M_ref[0] - mean) * rs * g_ref[0:1] + b_ref[0:1]
    o_ref[0] = jnp.where(xh >= 0, xh, 0.2 * xh)


def _finalize(M, st, g, b):
    B, N, O = M.shape
    NB = 512
    cnt = jnp.full((1,), float(B * N * K), jnp.float32)
    return pl.pallas_call(
        _fin_body,
        grid=(B, N // NB),
        in_specs=[
            pl.BlockSpec((1, NB, O), lambda bb, j: (bb, j, 0)),
            pl.BlockSpec((2, O), lambda bb, j: (0, 0)),
            pl.BlockSpec((1, O), lambda bb, j: (0, 0)),
            pl.BlockSpec((1, O), lambda bb, j: (0, 0)),
            pl.BlockSpec(memory_space=pltpu.SMEM),
        ],
        out_specs=pl.BlockSpec((1, NB, O), lambda bb, j: (bb, j, 0)),
        out_shape=jax.ShapeDtypeStruct((B, N, O), jnp.float32),
    )(M, st, g.reshape(1, O), b.reshape(1, O), cnt)


def _l5_body(cat_ref, W_ref, mx_ref, st_ref, s_acc, ss_acc, mx_acc):
    i = pl.program_id(0)
    j = pl.program_id(1)
    nb = pl.num_programs(1)

    @pl.when((i == 0) & (j == 0))
    def _init():
        s_acc[...] = jnp.zeros_like(s_acc)
        ss_acc[...] = jnp.zeros_like(ss_acc)

    @pl.when(j == 0)
    def _initmx():
        mx_acc[...] = jnp.full_like(mx_acc, NEG)

    h = jax.lax.dot_general(
        cat_ref[0].astype(jnp.bfloat16), W_ref[...].astype(jnp.bfloat16),
        (((1,), (1,)), ((), ())), preferred_element_type=jnp.float32)
    s_acc[...] += jnp.sum(h, axis=0, keepdims=True)
    ss_acc[...] += jnp.sum(h * h, axis=0, keepdims=True)
    mx_acc[...] = jnp.maximum(mx_acc[...], jnp.max(h, axis=0, keepdims=True))

    @pl.when(j == nb - 1)
    def _finmx():
        mx_ref[pl.ds(i, 1)] = mx_acc[...]

    @pl.when((i == pl.num_programs(0) - 1) & (j == nb - 1))
    def _fin():
        st_ref[0:1] = s_acc[...]
        st_ref[1:2] = ss_acc[...]


def _layer5(cat, W5):
    B, N, C = cat.shape
    O = W5.shape[0]
    NB = 512
    return pl.pallas_call(
        _l5_body,
        grid=(B, N // NB),
        in_specs=[
            pl.BlockSpec((1, NB, C), lambda b, j: (b, j, 0)),
            pl.BlockSpec((O, C), lambda b, j: (0, 0)),
        ],
        out_specs=[
            pl.BlockSpec((B, O), lambda b, j: (0, 0)),
            pl.BlockSpec((2, O), lambda b, j: (0, 0)),
        ],
        out_shape=[
            jax.ShapeDtypeStruct((B, O), jnp.float32),
            jax.ShapeDtypeStruct((2, O), jnp.float32),
        ],
        scratch_shapes=[
            pltpu.VMEM((1, O), jnp.float32),
            pltpu.VMEM((1, O), jnp.float32),
            pltpu.VMEM((1, O), jnp.float32),
        ],
    )(cat, W5)


def _emb_body(m5_ref, st_ref, g_ref, b_ref, cnt_ref, We_ref, o_ref):
    cnt = cnt_ref[0]
    mean = st_ref[0:1] / cnt
    var = st_ref[1:2] / cnt - mean * mean
    rs = jax.lax.rsqrt(var + EPS)
    xh = (m5_ref[...] - mean) * rs * g_ref[0:1] + b_ref[0:1]
    xg = jnp.where(xh >= 0, xh, 0.2 * xh)
    o_ref[...] = jax.lax.dot_general(
        xg.astype(jnp.bfloat16), We_ref[...].astype(jnp.bfloat16),
        (((1,), (1,)), ((), ())), preferred_element_type=jnp.float32)


def _embed(m5, st, g, b, Wemb, n_total):
    B, C = m5.shape
    O = Wemb.shape[0]
    cnt = jnp.full((1,), float(n_total), jnp.float32)
    return pl.pallas_call(
        _emb_body,
        in_specs=[
            pl.BlockSpec((B, C), lambda: (0, 0)),
            pl.BlockSpec((2, C), lambda: (0, 0)),
            pl.BlockSpec((1, C), lambda: (0, 0)),
            pl.BlockSpec((1, C), lambda: (0, 0)),
            pl.BlockSpec(memory_space=pltpu.SMEM),
            pl.BlockSpec((O, C), lambda: (0, 0)),
        ],
        out_specs=pl.BlockSpec((B, O), lambda: (0, 0)),
        out_shape=jax.ShapeDtypeStruct((B, O), jnp.float32),
    )(m5, st, g.reshape(1, C), b.reshape(1, C), cnt, Wemb)


def _edge_layer(xn, W, g, b):
    B, N, C = xn.shape
    idx = _knn_pallas(xn)
    gidx = (idx + (jnp.arange(B) * N)[:, None, None]).reshape(B * N * K)
    Cp = 128
    if C == Cp:
        x2d = xn.reshape(B * N, C)
    else:
        x2d = jnp.pad(xn.reshape(B * N, C), ((0, 0), (0, Cp - C)))
    G = _sc_gather(x2d, gidx, 256)
    M, st = _edge_conv(G.reshape(B, N, K, Cp), xn, W)
    return _finalize(M, st, g, b)


def kernel(x, W1, W2, W3, W4, W5, Wemb, g1, b1, g2, b2, g3, b3, g4, b4, g5, b5):
    x1 = _edge_layer(x, W1, g1, b1)
    x2 = _edge_layer(x1, W2, g2, b2)
    x3 = _edge_layer(x2, W3, g3, b3)
    x4 = _edge_layer(x3, W4, g4, b4)
    cat = jnp.concatenate([x1, x2, x3, x4], axis=-1)
    m5, st5 = _layer5(cat, W5)
    return _embed(m5, st5, g5, b5, Wemb, cat.shape[0] * cat.shape[1])
